# Initial kernel scaffold; baseline (speedup 1.0000x reference)
import jax
import jax.numpy as jnp
from jax.experimental import pallas as pl


def kernel(x_num, x_cat, tables, bn_gamma, bn_beta, W1, b1, W2, b2, W3, b3):
    raise NotImplementedError("write your pallas kernel here")


# speedup vs baseline: 7.9356x; 7.9356x over previous
import functools

import jax
import jax.numpy as jnp
from jax import lax
from jax.experimental import pallas as pl
from jax.experimental.pallas import tpu as pltpu
from jax.experimental.pallas import tpu_sc as plsc


def _make_sc_gather(n_chunks: int, chunk: int, d: int):
    info = plsc.get_sparse_core_info()
    nw = info.num_cores * info.num_subcores
    assert n_chunks % nw == 0
    r = n_chunks // nw
    k = 8
    assert r % k == 0
    g = r // k

    mesh = plsc.VectorSubcoreMesh(core_axis_name="c", subcore_axis_name="s")

    @functools.partial(
        pl.kernel,
        mesh=mesh,
        compiler_params=pltpu.CompilerParams(use_tc_tiling_on_sc=False),
        out_type=jax.ShapeDtypeStruct((n_chunks, chunk, d), jnp.float32),
        scratch_types=[
            pltpu.VMEM((r, chunk), jnp.int32),
            pltpu.VMEM((k, chunk, d), jnp.float32),
            pltpu.SemaphoreType.DMA,
        ],
    )
    def sc_gather(idx_hbm, tab_hbm, out_hbm, idx_v, rows_v, sem):
        wid = lax.axis_index("s") * info.num_cores + lax.axis_index("c")
        base = wid * r
        pltpu.sync_copy(idx_hbm.at[pl.ds(base, r)], idx_v)

        def body(gi, carry):
            cps = [
                pltpu.async_copy(
                    tab_hbm.at[idx_v.at[gi * k + j]], rows_v.at[j], sem)
                for j in range(k)
            ]
            for cp in cps:
                cp.wait()
            pltpu.sync_copy(rows_v, out_hbm.at[pl.ds(base + gi * k, k)])
            return carry

        lax.fori_loop(0, g, body, 0)

    return sc_gather


def _mlp_body(xnum_ref, cat_ref, gamma_ref, beta_ref, w1n_ref, w1c_ref,
              b1_ref, w2_ref, b2_ref, w3_ref, b3_ref, out_ref, *, bb: int):
    i = pl.program_id(0)
    xn_all = xnum_ref[...]
    mean = jnp.mean(xn_all, axis=0, keepdims=True)
    var = jnp.mean(jnp.square(xn_all - mean), axis=0, keepdims=True)
    inv = lax.rsqrt(var + 1e-5)
    xb = xnum_ref[pl.ds(i * bb, bb), :]
    xb = (xb - mean) * (inv * gamma_ref[...]) + beta_ref[...]
    h = jnp.dot(xb, w1n_ref[...], preferred_element_type=jnp.float32)
    h = h + jnp.dot(cat_ref[...], w1c_ref[...],
                    preferred_element_type=jnp.float32)
    h = jnp.maximum(h + b1_ref[...], 0.0)
    h = jnp.maximum(
        jnp.dot(h, w2_ref[...], preferred_element_type=jnp.float32)
        + b2_ref[...], 0.0)
    out_ref[...] = (jnp.dot(h, w3_ref[...], preferred_element_type=jnp.float32)
                    + b3_ref[...])


def kernel(x_num, x_cat, tables, bn_gamma, bn_beta, W1, b1, W2, b2, W3, b3):
    B, NUM = x_num.shape
    F, V, D = tables.shape
    H = W2.shape[0]
    FD = F * D

    offs = (jnp.arange(F, dtype=jnp.int32) * V)[None, :]
    glob_idx = x_cat + offs
    chunk = 128
    n_chunks = (B * F) // chunk
    idx2d = glob_idx.reshape(n_chunks, chunk)
    tab2d = tables.reshape(F * V, D)

    rows = _make_sc_gather(n_chunks, chunk, D)(idx2d, tab2d)
    cat_out = rows.reshape(B, FD)

    bb = 2048
    grid = (B // bb,)
    out = pl.pallas_call(
        functools.partial(_mlp_body, bb=bb),
        grid=grid,
        in_specs=[
            pl.BlockSpec((B, NUM), lambda i: (0, 0)),
            pl.BlockSpec((bb, FD), lambda i: (i, 0)),
            pl.BlockSpec((1, NUM), lambda i: (0, 0)),
            pl.BlockSpec((1, NUM), lambda i: (0, 0)),
            pl.BlockSpec((NUM, H), lambda i: (0, 0)),
            pl.BlockSpec((FD, H), lambda i: (0, 0)),
            pl.BlockSpec((1, H), lambda i: (0, 0)),
            pl.BlockSpec((H, H), lambda i: (0, 0)),
            pl.BlockSpec((1, H), lambda i: (0, 0)),
            pl.BlockSpec((H, 1), lambda i: (0, 0)),
            pl.BlockSpec((1, 1), lambda i: (0, 0)),
        ],
        out_specs=pl.BlockSpec((bb, 1), lambda i: (i, 0)),
        out_shape=jax.ShapeDtypeStruct((B, 1), jnp.float32),
        compiler_params=pltpu.CompilerParams(
            dimension_semantics=("arbitrary",)),
    )(x_num, cat_out, bn_gamma.reshape(1, NUM), bn_beta.reshape(1, NUM),
      W1[:NUM], W1[NUM:], b1.reshape(1, H), W2, b2.reshape(1, H),
      W3, b3.reshape(1, 1))
    return out


# speedup vs baseline: 8.3277x; 1.0494x over previous
import functools

import jax
import jax.numpy as jnp
from jax import lax
from jax.experimental import pallas as pl
from jax.experimental.pallas import tpu as pltpu
from jax.experimental.pallas import tpu_sc as plsc

_C = 128
_Q = 8


def _make_sc_plane_gather(f: int, d: int, v: int, b: int):
    info = plsc.get_sparse_core_info()
    nw = info.num_cores * info.num_subcores
    n_planes = f * d
    assert n_planes % nw == 0 and b % _C == 0
    ppw = n_planes // nw
    nc = b // _C

    mesh = plsc.VectorSubcoreMesh(core_axis_name="c", subcore_axis_name="s")

    @functools.partial(
        pl.kernel,
        mesh=mesh,
        compiler_params=pltpu.CompilerParams(use_tc_tiling_on_sc=False),
        out_type=jax.ShapeDtypeStruct((n_planes // 8, nc, 8, _C),
                                      jnp.float32),
        scratch_types=[
            pltpu.VMEM((nc, _C), jnp.int32),
            pltpu.VMEM((nc, _C), jnp.float32),
            pltpu.VMEM((nc, _C), jnp.float32),
            pltpu.SemaphoreType.DMA,
            pltpu.SemaphoreType.DMA,
            pltpu.SemaphoreType.DMA,
        ],
    )
    def sc_gather(idx_hbm, tab_hbm, out_hbm, idx_v, out_a, out_b, gsem,
                  wsem_a, wsem_b):
        wid = lax.axis_index("s") * info.num_cores + lax.axis_index("c")
        base = wid * ppw
        bufs = (out_a, out_b)
        wsems = (wsem_a, wsem_b)
        pend = [None, None]
        for j in range(ppw):
            p = base + j
            row = tab_hbm.at[p]
            if j == 0:
                pltpu.sync_copy(idx_hbm.at[p // d], idx_v)
            else:
                @pl.when(p % d == 0)
                def _():
                    pltpu.sync_copy(idx_hbm.at[p // d], idx_v)
            buf = bufs[j % 2]
            if pend[j % 2] is not None:
                pend[j % 2].wait()

            def fire_drain(c, carry, row=row, buf=buf):
                pltpu.async_copy(row.at[idx_v.at[c]], buf.at[c], gsem)

                @pl.when(c >= _Q)
                def _():
                    pltpu.make_async_copy(
                        row.at[idx_v.at[c - _Q]], buf.at[c - _Q], gsem).wait()
                return carry

            lax.fori_loop(0, nc, fire_drain, 0)

            def drain(c, carry, row=row, buf=buf):
                pltpu.make_async_copy(
                    row.at[idx_v.at[c]], buf.at[c], gsem).wait()
                return carry

            lax.fori_loop(nc - _Q, nc, drain, 0)
            pend[j % 2] = pltpu.async_copy(
                buf, out_hbm.at[p // 8, :, p % 8, :], wsems[j % 2])
        for cp in pend:
            if cp is not None:
                cp.wait()

    return sc_gather


def _mlp_t_body(xn_ref, cat_ref, gamma_ref, beta_ref, w1n_ref, w1c_ref,
                b1_ref, w2_ref, b2_ref, w3_ref, b3_ref, out_ref, *, bb: int):
    i = pl.program_id(0)
    xn = xn_ref[...]
    mean = jnp.mean(xn, axis=1, keepdims=True)
    var = jnp.mean(jnp.square(xn - mean), axis=1, keepdims=True)
    inv = lax.rsqrt(var + 1e-5)
    xb = xn_ref[:, pl.ds(i * bb, bb)]
    xb = (xb - mean) * (inv * gamma_ref[...]) + beta_ref[...]
    h = jnp.dot(w1n_ref[...], xb, preferred_element_type=jnp.float32)
    h = h + jnp.dot(w1c_ref[...], cat_ref[...],
                    preferred_element_type=jnp.float32)
    h = jnp.maximum(h + b1_ref[...], 0.0)
    h = jnp.maximum(
        jnp.dot(w2_ref[...], h, preferred_element_type=jnp.float32)
        + b2_ref[...], 0.0)
    out_ref[...] = (jnp.dot(w3_ref[...], h, preferred_element_type=jnp.float32)
                    + b3_ref[...])


def kernel(x_num, x_cat, tables, bn_gamma, bn_beta, W1, b1, W2, b2, W3, b3):
    B, NUM = x_num.shape
    F, V, D = tables.shape
    H = W2.shape[0]
    FD = F * D

    xnT = x_num.T
    idxT = x_cat.T.reshape(F, B // _C, _C)
    planes = tables.transpose(0, 2, 1).reshape(FD, V)

    catT4 = _make_sc_plane_gather(F, D, V, B)(idxT, planes)
    catT = catT4.transpose(0, 2, 1, 3).reshape(FD, B)

    bb = 2048
    grid = (B // bb,)
    outT = pl.pallas_call(
        functools.partial(_mlp_t_body, bb=bb),
        grid=grid,
        in_specs=[
            pl.BlockSpec((NUM, B), lambda i: (0, 0)),
            pl.BlockSpec((FD, bb), lambda i: (0, i)),
            pl.BlockSpec((NUM, 1), lambda i: (0, 0)),
            pl.BlockSpec((NUM, 1), lambda i: (0, 0)),
            pl.BlockSpec((H, NUM), lambda i: (0, 0)),
            pl.BlockSpec((H, FD), lambda i: (0, 0)),
            pl.BlockSpec((H, 1), lambda i: (0, 0)),
            pl.BlockSpec((H, H), lambda i: (0, 0)),
            pl.BlockSpec((H, 1), lambda i: (0, 0)),
            pl.BlockSpec((1, H), lambda i: (0, 0)),
            pl.BlockSpec((1, 1), lambda i: (0, 0)),
        ],
        out_specs=pl.BlockSpec((1, bb), lambda i: (0, i)),
        out_shape=jax.ShapeDtypeStruct((1, B), jnp.float32),
        compiler_params=pltpu.CompilerParams(
            dimension_semantics=("arbitrary",)),
    )(xnT, catT, bn_gamma.reshape(NUM, 1), bn_beta.reshape(NUM, 1),
      W1[:NUM].T, W1[NUM:].T, b1.reshape(H, 1), W2.T, b2.reshape(H, 1),
      W3.T, b3.reshape(1, 1))
    return outT.reshape(B, 1)


# speedup vs baseline: 9.9048x; 1.1894x over previous
import functools

import jax
import jax.numpy as jnp
from jax import lax
from jax.experimental import pallas as pl
from jax.experimental.pallas import tpu as pltpu
from jax.experimental.pallas import tpu_sc as plsc

_C = 2048
_Q = 4
_L = 128


def _make_sc_plane_gather(f: int, d: int, v: int, b: int):
    info = plsc.get_sparse_core_info()
    nw = info.num_cores * info.num_subcores
    n_planes = f * d
    assert n_planes % nw == 0 and b % _C == 0
    ppw = n_planes // nw
    nc = b // _C

    mesh = plsc.VectorSubcoreMesh(core_axis_name="c", subcore_axis_name="s")

    @functools.partial(
        pl.kernel,
        mesh=mesh,
        compiler_params=pltpu.CompilerParams(use_tc_tiling_on_sc=False),
        out_type=jax.ShapeDtypeStruct((n_planes, b), jnp.float32),
        scratch_types=[
            pltpu.VMEM((nc, _C), jnp.int32),
            pltpu.VMEM((b,), jnp.float32),
            pltpu.VMEM((b,), jnp.float32),
            pltpu.SemaphoreType.DMA,
            pltpu.SemaphoreType.DMA,
            pltpu.SemaphoreType.DMA,
        ],
    )
    def sc_gather(idx_hbm, tab_hbm, out_hbm, idx_v, out_a, out_b, gsem,
                  wsem_a, wsem_b):
        wid = lax.axis_index("s") * info.num_cores + lax.axis_index("c")
        base = wid * ppw
        bufs = (out_a, out_b)
        wsems = (wsem_a, wsem_b)
        pend = [None, None]
        for j in range(ppw):
            p = base + j
            row = tab_hbm.at[p]
            if j == 0:
                pltpu.sync_copy(idx_hbm.at[p // d], idx_v)
            else:
                @pl.when(p % d == 0)
                def _():
                    pltpu.sync_copy(idx_hbm.at[p // d], idx_v)
            buf = bufs[j % 2]
            if pend[j % 2] is not None:
                pend[j % 2].wait()

            def fire_drain(c, carry, row=row, buf=buf):
                pltpu.async_copy(
                    row.at[idx_v.at[c]], buf.at[pl.ds(c * _C, _C)], gsem)

                @pl.when(c >= _Q)
                def _():
                    pltpu.make_async_copy(
                        row.at[idx_v.at[c - _Q]],
                        buf.at[pl.ds((c - _Q) * _C, _C)], gsem).wait()
                return carry

            lax.fori_loop(0, nc, fire_drain, 0)

            def drain(c, carry, row=row, buf=buf):
                pltpu.make_async_copy(
                    row.at[idx_v.at[c]], buf.at[pl.ds(c * _C, _C)],
                    gsem).wait()
                return carry

            lax.fori_loop(nc - _Q, nc, drain, 0)
            pend[j % 2] = pltpu.async_copy(buf, out_hbm.at[p], wsems[j % 2])
        for cp in pend:
            if cp is not None:
                cp.wait()

    return sc_gather


def _mlp_t_body(xn_ref, cat_ref, gamma_ref, beta_ref, w1n_ref, w1c_ref,
                b1_ref, w2_ref, b2_ref, w3_ref, b3_ref, out_ref, *, bb: int):
    i = pl.program_id(0)
    xn = xn_ref[...]
    mean = jnp.mean(xn, axis=1, keepdims=True)
    var = jnp.mean(jnp.square(xn - mean), axis=1, keepdims=True)
    inv = lax.rsqrt(var + 1e-5)
    xb = xn_ref[:, pl.ds(i * bb, bb)]
    xb = (xb - mean) * (inv * gamma_ref[...]) + beta_ref[...]
    h = jnp.dot(w1n_ref[...], xb, preferred_element_type=jnp.float32)
    w1c = w1c_ref[...]
    hc = [jnp.dot(w1c, cat_ref[:, c, :], preferred_element_type=jnp.float32)
          for c in range(bb // _L)]
    h = h + jnp.concatenate(hc, axis=1)
    h = jnp.maximum(h + b1_ref[...], 0.0)
    h = jnp.maximum(
        jnp.dot(w2_ref[...], h, preferred_element_type=jnp.float32)
        + b2_ref[...], 0.0)
    out_ref[...] = (jnp.dot(w3_ref[...], h, preferred_element_type=jnp.float32)
                    + b3_ref[...])


def kernel(x_num, x_cat, tables, bn_gamma, bn_beta, W1, b1, W2, b2, W3, b3):
    B, NUM = x_num.shape
    F, V, D = tables.shape
    H = W2.shape[0]
    FD = F * D

    xnT = x_num.T
    idxT = x_cat.T.reshape(F, B // _C, _C)
    planes = tables.transpose(0, 2, 1).reshape(FD, V)

    catT = _make_sc_plane_gather(F, D, V, B)(idxT, planes)
    cat3 = catT.reshape(FD, B // _L, _L)

    bb = 2048
    grid = (B // bb,)
    outT = pl.pallas_call(
        functools.partial(_mlp_t_body, bb=bb),
        grid=grid,
        in_specs=[
            pl.BlockSpec((NUM, B), lambda i: (0, 0)),
            pl.BlockSpec((FD, bb // _L, _L), lambda i: (0, i, 0)),
            pl.BlockSpec((NUM, 1), lambda i: (0, 0)),
            pl.BlockSpec((NUM, 1), lambda i: (0, 0)),
            pl.BlockSpec((H, NUM), lambda i: (0, 0)),
            pl.BlockSpec((H, FD), lambda i: (0, 0)),
            pl.BlockSpec((H, 1), lambda i: (0, 0)),
            pl.BlockSpec((H, H), lambda i: (0, 0)),
            pl.BlockSpec((H, 1), lambda i: (0, 0)),
            pl.BlockSpec((1, H), lambda i: (0, 0)),
            pl.BlockSpec((1, 1), lambda i: (0, 0)),
        ],
        out_specs=pl.BlockSpec((1, bb), lambda i: (0, i)),
        out_shape=jax.ShapeDtypeStruct((1, B), jnp.float32),
        compiler_params=pltpu.CompilerParams(
            dimension_semantics=("arbitrary",)),
    )(xnT, cat3, bn_gamma.reshape(NUM, 1), bn_beta.reshape(NUM, 1),
      W1[:NUM].T, W1[NUM:].T, b1.reshape(H, 1), W2.T, b2.reshape(H, 1),
      W3.T, b3.reshape(1, 1))
    return outT.reshape(B, 1)
